# baseline (speedup 1.0000x reference)
import jax
import jax.numpy as jnp
from jax.experimental import pallas as pl
from jax.experimental.pallas import tpu as pltpu

_B, _S, _D = 4, 4096, 2048
_T = 3


def _probe(x_ref, emb_ref, out_ref, sem_x, sem_e):
    copies = []
    for b in range(_B):
        cx = pltpu.make_async_copy(
            x_ref.at[b], out_ref.at[b, pl.ds(0, _S)], sem_x)
        cx.start()
        ce = pltpu.make_async_copy(
            emb_ref, out_ref.at[b, pl.ds(_S, _T)], sem_e)
        ce.start()
        copies.append((cx, ce))
    for cx, ce in copies:
        cx.wait()
        ce.wait()


def kernel(x, embedding):
    return pl.pallas_call(
        _probe,
        out_shape=jax.ShapeDtypeStruct((_B, _S + _T, _D), x.dtype),
        in_specs=[
            pl.BlockSpec(memory_space=pltpu.MemorySpace.HBM),
            pl.BlockSpec(memory_space=pltpu.MemorySpace.HBM),
        ],
        out_specs=pl.BlockSpec(memory_space=pltpu.MemorySpace.HBM),
        scratch_shapes=[pltpu.SemaphoreType.DMA, pltpu.SemaphoreType.DMA],
    )(x, embedding)


# speedup vs baseline: 14.0992x; 14.0992x over previous
import functools

import jax
import jax.numpy as jnp
from jax import lax
from jax.experimental import pallas as pl
from jax.experimental.pallas import tpu as pltpu
from jax.experimental.pallas import tpu_sc as plsc

_B, _S, _D = 4, 4096, 2048
_T = 3
_NW = 32
_WPB = _NW // _B
_CH = 16
_NCHB = 255
_NI = 32


def _sc_concat(x_hbm, emb_hbm, out_hbm, buf0, buf1, ebuf, sem_r, sem_w):
    c = lax.axis_index("c")
    s = lax.axis_index("s")
    wid = s * 2 + c
    b = wid // _WPB
    wk = wid % _WPB
    bufs = (buf0, buf1)
    lanes = lax.iota(jnp.int32, _CH)

    @pl.when(wk == 0)
    def _():
        pltpu.sync_copy(emb_hbm, ebuf.at[pl.ds(0, _T)])
        pltpu.sync_copy(ebuf.at[pl.ds(0, _T)], out_hbm.at[b, pl.ds(0, _T)])
        pltpu.sync_copy(x_hbm.at[b, pl.ds(0, _CH)], ebuf)
        pltpu.async_copy(ebuf, out_hbm.at[b].at[_T + lanes], sem_w)
        pltpu.make_async_copy(x_hbm.at[b, pl.ds(0, _CH)], ebuf, sem_w).wait()

    @pl.when(wk == 1)
    def _():
        pltpu.sync_copy(x_hbm.at[b, pl.ds(_S - _CH, _CH)], ebuf)
        pltpu.async_copy(ebuf, out_hbm.at[b].at[(_S - _CH + _T) + lanes],
                         sem_w)
        pltpu.make_async_copy(x_hbm.at[b, pl.ds(0, _CH)], ebuf, sem_w).wait()

    def valid(i):
        return (i * _WPB + wk) < _NCHB

    def gather(i, buf):
        cc = i * _WPB + wk
        pltpu.async_copy(x_hbm.at[b].at[(5 + _CH * cc) + lanes], buf, sem_r)

    def write(i, buf):
        cc = i * _WPB + wk
        pltpu.async_copy(buf, out_hbm.at[b, pl.ds(8 + _CH * cc, _CH)], sem_w)

    def wait_one(sem):
        pltpu.make_async_copy(x_hbm.at[b, pl.ds(0, _CH)], buf0, sem).wait()

    gather(0, bufs[0])

    def body(i2, carry):
        for j in range(2):
            i = i2 * 2 + j
            cur = bufs[j]
            nxt = bufs[1 - j]

            @pl.when(valid(i))
            def _():
                wait_one(sem_r)
                write(i, cur)

            @pl.when((i >= 1) & valid(i))
            def _():
                wait_one(sem_w)

            @pl.when(valid(i + 1))
            def _():
                gather(i + 1, nxt)
        return carry

    lax.fori_loop(0, _NI // 2, body, 0)
    wait_one(sem_w)


def kernel(x, embedding):
    mesh = plsc.VectorSubcoreMesh(core_axis_name="c", subcore_axis_name="s")
    k = functools.partial(
        pl.kernel,
        mesh=mesh,
        out_type=jax.ShapeDtypeStruct((_B, _S + _T, _D), x.dtype),
        scratch_types=[
            pltpu.VMEM((_CH, _D), jnp.float32),
            pltpu.VMEM((_CH, _D), jnp.float32),
            pltpu.VMEM((_CH, _D), jnp.float32),
            pltpu.SemaphoreType.DMA,
            pltpu.SemaphoreType.DMA,
        ],
    )(_sc_concat)
    return k(x, embedding)


# speedup vs baseline: 15.7146x; 1.1146x over previous
import jax
import jax.numpy as jnp
from jax.experimental import pallas as pl
from jax.experimental.pallas import tpu as pltpu

_B, _S, _D = 4, 4096, 2048
_T = 3
_ROWS = 512
_NCHB = _S // _ROWS
_NBUF = 4


def _tc_kernel(x_ref, emb_ref, out_ref, inb, outb, carry, tailb,
               sem_in, sem_out, sem_t):
    chunks = [(b, j) for b in range(_B) for j in range(_NCHB)]
    n = len(chunks)

    def in_copy(i):
        b, j = chunks[i]
        k = i % _NBUF
        return pltpu.make_async_copy(
            x_ref.at[b, pl.ds(j * _ROWS, _ROWS)], inb.at[k], sem_in.at[k])

    def out_copy(i):
        b, j = chunks[i]
        k = i % _NBUF
        return pltpu.make_async_copy(
            outb.at[k], out_ref.at[b, pl.ds(j * _ROWS, _ROWS)], sem_out.at[k])

    for i in range(_NBUF):
        in_copy(i).start()

    for i in range(n):
        b, j = chunks[i]
        k = i % _NBUF
        in_copy(i).wait()
        if i >= _NBUF:
            out_copy(i - _NBUF).wait()
        if j == 0:
            outb[k, 0:_T] = emb_ref[...]
        else:
            outb[k, 0:_T] = carry[0:_T]
        outb[k, _T:_ROWS] = inb[k, 0:_ROWS - _T]
        carry[0:_T] = inb[k, _ROWS - _T:_ROWS]
        if j == _NCHB - 1:
            tailb[8 * b:8 * b + _T] = inb[k, _ROWS - _T:_ROWS]
            pltpu.make_async_copy(
                tailb.at[pl.ds(8 * b, _T)],
                out_ref.at[b, pl.ds(_S, _T)], sem_t.at[b]).start()
        if i + _NBUF < n:
            in_copy(i + _NBUF).start()
        out_copy(i).start()

    for i in range(n - _NBUF, n):
        out_copy(i).wait()
    for b in range(_B):
        pltpu.make_async_copy(
            tailb.at[pl.ds(8 * b, _T)],
            out_ref.at[b, pl.ds(_S, _T)], sem_t.at[b]).wait()


def kernel(x, embedding):
    return pl.pallas_call(
        _tc_kernel,
        out_shape=jax.ShapeDtypeStruct((_B, _S + _T, _D), x.dtype),
        in_specs=[
            pl.BlockSpec(memory_space=pltpu.MemorySpace.HBM),
            pl.BlockSpec(memory_space=pltpu.MemorySpace.VMEM),
        ],
        out_specs=pl.BlockSpec(memory_space=pltpu.MemorySpace.HBM),
        scratch_shapes=[
            pltpu.VMEM((_NBUF, _ROWS, _D), x.dtype),
            pltpu.VMEM((_NBUF, _ROWS, _D), x.dtype),
            pltpu.VMEM((8, _D), x.dtype),
            pltpu.VMEM((32, _D), x.dtype),
            pltpu.SemaphoreType.DMA((_NBUF,)),
            pltpu.SemaphoreType.DMA((_NBUF,)),
            pltpu.SemaphoreType.DMA((_B,)),
        ],
    )(x, embedding)
